# baseline (speedup 1.0000x reference)
import numpy as np
import jax
import jax.numpy as jnp
from jax.experimental import pallas as pl
from jax.experimental.pallas import tpu as pltpu

_PATCH = 16
_HEIGHT = 512
_WIDTH = 512
_DEPTH = 128
_EMBED = 768


def _axis_positions(axis_num):
    n = axis_num // _PATCH
    idx = np.arange(n, dtype=np.float64)
    frm = np.round(idx * _PATCH / axis_num * _DEPTH).astype(np.int32)
    to = np.round((idx + 1) * _PATCH / axis_num * _DEPTH).astype(np.int32)
    return np.round((frm + to).astype(np.float64) / 2.0).astype(np.int32)


_ROW_AXIS = _axis_positions(_HEIGHT)
_COL_AXIS = _axis_positions(_WIDTH)
_NROWS = _HEIGHT // _PATCH
_NCOLS = _WIDTH // _PATCH
_SEQ = _NROWS * _NCOLS

_NBUF = 8
_CROWS = 1024
_REPS = _CROWS // _SEQ


def _ring_kernel(x_hbm, row_hbm, col_hbm, o_hbm,
                 row_v, col_v, enc_v, in_bufs, out_bufs, tsem, rsem, wsem):
    n_rows = x_hbm.shape[0]
    n_chunks = n_rows // _CROWS

    pltpu.make_async_copy(row_hbm, row_v, tsem).start()
    pltpu.make_async_copy(row_hbm, row_v, tsem).wait()
    pltpu.make_async_copy(col_hbm, col_v, tsem).start()
    pltpu.make_async_copy(col_hbm, col_v, tsem).wait()
    row_rows = jnp.concatenate(
        [row_v[int(p)][None, :] for p in _ROW_AXIS], axis=0
    )
    col_rows = jnp.concatenate(
        [col_v[int(p)][None, :] for p in _COL_AXIS], axis=0
    )
    enc_v[...] = (row_rows[:, None, :] + col_rows[None, :, :]).reshape(_SEQ, _EMBED)

    def _read(i, slot):
        pltpu.make_async_copy(
            x_hbm.at[pl.ds(i * _CROWS, _CROWS), :], in_bufs.at[slot], rsem.at[slot]
        ).start()

    for j in range(_NBUF):
        _read(j, j)

    for i in range(n_chunks):
        slot = i % _NBUF
        if i >= _NBUF:
            pltpu.make_async_copy(
                out_bufs.at[slot],
                o_hbm.at[pl.ds((i - _NBUF) * _CROWS, _CROWS), :],
                wsem.at[slot],
            ).wait()
        pltpu.make_async_copy(
            x_hbm.at[pl.ds(i * _CROWS, _CROWS), :], in_bufs.at[slot], rsem.at[slot]
        ).wait()
        for r in range(_REPS):
            sl = pl.ds(r * _SEQ, _SEQ)
            out_bufs[slot, sl, :] = in_bufs[slot, sl, :] + enc_v[...]
        pltpu.make_async_copy(
            out_bufs.at[slot], o_hbm.at[pl.ds(i * _CROWS, _CROWS), :], wsem.at[slot]
        ).start()
        if i + _NBUF < n_chunks:
            _read(i + _NBUF, slot)

    for i in range(n_chunks - _NBUF, n_chunks):
        slot = i % _NBUF
        pltpu.make_async_copy(
            out_bufs.at[slot], o_hbm.at[pl.ds(i * _CROWS, _CROWS), :], wsem.at[slot]
        ).wait()


def kernel(inputs, row_embedding, col_embedding):
    B, S, E = inputs.shape
    flat = inputs.reshape(B * S, E)
    out = pl.pallas_call(
        _ring_kernel,
        in_specs=[
            pl.BlockSpec(memory_space=pl.ANY),
            pl.BlockSpec(memory_space=pl.ANY),
            pl.BlockSpec(memory_space=pl.ANY),
        ],
        out_specs=pl.BlockSpec(memory_space=pl.ANY),
        out_shape=jax.ShapeDtypeStruct((B * S, E), inputs.dtype),
        scratch_shapes=[
            pltpu.VMEM((_DEPTH, E), jnp.float32),
            pltpu.VMEM((_DEPTH, E), jnp.float32),
            pltpu.VMEM((_SEQ, E), jnp.float32),
            pltpu.VMEM((_NBUF, _CROWS, E), jnp.float32),
            pltpu.VMEM((_NBUF, _CROWS, E), jnp.float32),
            pltpu.SemaphoreType.DMA,
            pltpu.SemaphoreType.DMA((_NBUF,)),
            pltpu.SemaphoreType.DMA((_NBUF,)),
        ],
        compiler_params=pltpu.CompilerParams(
            vmem_limit_bytes=128 * 1024 * 1024,
        ),
    )(flat, row_embedding, col_embedding)
    return out.reshape(B, S, E)


# speedup vs baseline: 1.0446x; 1.0446x over previous
import numpy as np
import jax
import jax.numpy as jnp
from jax.experimental import pallas as pl
from jax.experimental.pallas import tpu as pltpu

_PATCH = 16
_HEIGHT = 512
_WIDTH = 512
_DEPTH = 128
_EMBED = 768


def _axis_positions(axis_num):
    n = axis_num // _PATCH
    idx = np.arange(n, dtype=np.float64)
    frm = np.round(idx * _PATCH / axis_num * _DEPTH).astype(np.int32)
    to = np.round((idx + 1) * _PATCH / axis_num * _DEPTH).astype(np.int32)
    return np.round((frm + to).astype(np.float64) / 2.0).astype(np.int32)


_ROW_AXIS = _axis_positions(_HEIGHT)
_COL_AXIS = _axis_positions(_WIDTH)
_NROWS = _HEIGHT // _PATCH
_NCOLS = _WIDTH // _PATCH


def _add_kernel(x_ref, row_ref, col_ref, o_ref, enc_ref):
    @pl.when(pl.program_id(0) == 0)
    def _():
        row_rows = jnp.concatenate(
            [row_ref[int(p)][None, :] for p in _ROW_AXIS], axis=0
        )
        col_rows = jnp.concatenate(
            [col_ref[int(p)][None, :] for p in _COL_AXIS], axis=0
        )
        enc = row_rows[:, None, :] + col_rows[None, :, :]
        enc_ref[...] = enc.reshape(_NROWS * _NCOLS, _EMBED)

    o_ref[...] = x_ref[...] + enc_ref[...][None, :, :]


_BB = 4


def kernel(inputs, row_embedding, col_embedding):
    B, S, E = inputs.shape
    return pl.pallas_call(
        _add_kernel,
        grid=(B // _BB,),
        in_specs=[
            pl.BlockSpec((_BB, S, E), lambda b: (b, 0, 0)),
            pl.BlockSpec((_DEPTH, E), lambda b: (0, 0)),
            pl.BlockSpec((_DEPTH, E), lambda b: (0, 0)),
        ],
        out_specs=pl.BlockSpec((_BB, S, E), lambda b: (b, 0, 0)),
        out_shape=jax.ShapeDtypeStruct((B, S, E), inputs.dtype),
        scratch_shapes=[pltpu.VMEM((S, E), jnp.float32)],
        compiler_params=pltpu.CompilerParams(
            vmem_limit_bytes=128 * 1024 * 1024,
        ),
    )(inputs, row_embedding, col_embedding)
